# baseline (speedup 1.0000x reference)
import jax
import jax.numpy as jnp
from jax.experimental import pallas as pl
from jax.experimental.pallas import tpu as pltpu

NUM_EMB = 1024
DIM = 64
CCOST = 0.25
ROWS = 16 * 576
BR = 512
NBLK = ROWS // BR


def _vq_body(z_ref, cbt_ref, q_ref, idx_ref, loss_ref):
    i = pl.program_id(0)
    zb = z_ref[...]
    cbt = cbt_ref[...]
    c2 = jnp.sum(cbt * cbt, axis=0)
    z2 = jnp.sum(zb * zb, axis=1, keepdims=True)
    m = jax.lax.dot_general(zb, cbt, (((1,), (0,)), ((), ())),
                            preferred_element_type=jnp.float32)
    scores = (z2 - 2.0 * m) + c2[None, :]
    dmin = jnp.min(scores, axis=1)
    cols = jax.lax.broadcasted_iota(jnp.int32, scores.shape, 1)
    idx = jnp.min(jnp.where(scores == dmin[:, None], cols, NUM_EMB),
                  axis=1).astype(jnp.int32)
    onehot = (cols == idx[:, None]).astype(jnp.float32)
    q = jax.lax.dot_general(onehot, cbt, (((1,), (1,)), ((), ())),
                            preferred_element_type=jnp.float32,
                            precision=jax.lax.Precision.HIGHEST)
    q_ref[...] = q
    idx_ref[0, 0, :] = idx
    part = jnp.sum(dmin)

    @pl.when(i == 0)
    def _init():
        loss_ref[0, 0] = 0.0

    loss_ref[0, 0] += part


def kernel(z, codebook):
    flat_z = z.reshape(ROWS, DIM)
    cbt = codebook.T
    q, idx, loss_sum = pl.pallas_call(
        _vq_body,
        grid=(NBLK,),
        in_specs=[
            pl.BlockSpec((BR, DIM), lambda i: (i, 0)),
            pl.BlockSpec((DIM, NUM_EMB), lambda i: (0, 0)),
        ],
        out_specs=[
            pl.BlockSpec((BR, DIM), lambda i: (i, 0)),
            pl.BlockSpec((1, 1, BR), lambda i: (i, 0, 0)),
            pl.BlockSpec(memory_space=pltpu.SMEM,
                         block_shape=(1, 1), index_map=lambda i: (0, 0)),
        ],
        out_shape=[
            jax.ShapeDtypeStruct((ROWS, DIM), jnp.float32),
            jax.ShapeDtypeStruct((NBLK, 1, BR), jnp.int32),
            jax.ShapeDtypeStruct((1, 1), jnp.float32),
        ],
    )(flat_z, cbt)
    quantized = q.reshape(z.shape)
    indices = idx.reshape(z.shape[0], -1)
    loss = (1.0 + CCOST) * loss_sum[0, 0] / (ROWS * DIM)
    return quantized, indices, loss


# speedup vs baseline: 1.0915x; 1.0915x over previous
import functools

import jax
import jax.numpy as jnp
from jax import lax
from jax.experimental import pallas as pl
from jax.experimental.pallas import tpu as pltpu
from jax.experimental.pallas import tpu_sc as plsc

NUM_EMB = 1024
DIM = 64
CCOST = 0.25
ROWS = 16 * 576
BR = 512
NBLK = ROWS // BR

_info = plsc.get_sparse_core_info()
_NC, _NS = _info.num_cores, _info.num_subcores
NW = _NC * _NS
BPW = ROWS // NW


def _vq_body(z_ref, cbt_ref, idx_ref, loss_ref):
    i = pl.program_id(0)
    zb = z_ref[...]
    cbt = cbt_ref[...]
    c2 = jnp.sum(cbt * cbt, axis=0)
    z2 = jnp.sum(zb * zb, axis=1, keepdims=True)
    m = jax.lax.dot_general(zb, cbt, (((1,), (0,)), ((), ())),
                            preferred_element_type=jnp.float32)
    scores = (z2 - 2.0 * m) + c2[None, :]
    dmin = jnp.min(scores, axis=1)
    cols = lax.broadcasted_iota(jnp.int32, scores.shape, 1)
    idx = jnp.min(jnp.where(scores == dmin[:, None], cols, NUM_EMB),
                  axis=1).astype(jnp.int32)
    idx_ref[0, 0, :] = idx
    part = jnp.sum(dmin)

    @pl.when(i == 0)
    def _init():
        loss_ref[0, 0] = 0.0

    loss_ref[0, 0] += part


_sc_mesh = plsc.VectorSubcoreMesh(core_axis_name="c", subcore_axis_name="s")


PADW = 128


@functools.partial(
    pl.kernel, mesh=_sc_mesh,
    out_type=jax.ShapeDtypeStruct((ROWS, PADW), jnp.float32),
    scratch_types=[
        pltpu.VMEM((BPW,), jnp.int32),
        pltpu.VMEM((BPW, PADW), jnp.float32),
        pltpu.SemaphoreType.DMA,
    ],
)
def _sc_gather(cb_hbm, idx_hbm, out_hbm, idx_v, rows_v, sem):
    wid = lax.axis_index("s") * _NC + lax.axis_index("c")
    base = wid * BPW
    pltpu.sync_copy(idx_hbm.at[pl.ds(base, BPW)], idx_v)
    pltpu.async_copy(cb_hbm.at[idx_v], rows_v, sem).wait()
    pltpu.sync_copy(rows_v, out_hbm.at[pl.ds(base, BPW)])


def kernel(z, codebook):
    flat_z = z.reshape(ROWS, DIM)
    cbt = codebook.T
    idx, loss_sum = pl.pallas_call(
        _vq_body,
        grid=(NBLK,),
        in_specs=[
            pl.BlockSpec((BR, DIM), lambda i: (i, 0)),
            pl.BlockSpec((DIM, NUM_EMB), lambda i: (0, 0)),
        ],
        out_specs=[
            pl.BlockSpec((1, 1, BR), lambda i: (i, 0, 0)),
            pl.BlockSpec(memory_space=pltpu.SMEM,
                         block_shape=(1, 1), index_map=lambda i: (0, 0)),
        ],
        out_shape=[
            jax.ShapeDtypeStruct((NBLK, 1, BR), jnp.int32),
            jax.ShapeDtypeStruct((1, 1), jnp.float32),
        ],
    )(flat_z, cbt)
    idx_flat = idx.reshape(ROWS)
    cb_pad = jnp.pad(codebook, ((0, 0), (0, PADW - DIM)))
    q = _sc_gather(cb_pad, idx_flat)
    quantized = q[:, :DIM].reshape(z.shape)
    indices = idx_flat.reshape(z.shape[0], -1)
    loss = (1.0 + CCOST) * loss_sum[0, 0] / (ROWS * DIM)
    return quantized, indices, loss
